# baseline (speedup 1.0000x reference)
import functools

import jax
import jax.numpy as jnp
from jax import lax
from jax.experimental import pallas as pl
from jax.experimental.pallas import tpu as pltpu
from jax.experimental.pallas import tpu_sc as plsc

RES = 128
DENSITY_THRESHOLD = 0.01
N_POINTS = 2097152
N_WORDS = RES ** 3 // 32

N_WORKERS = 32
PTS_PER_WORKER = N_POINTS // N_WORKERS
CHUNK = 4096
N_CHUNKS = PTS_PER_WORKER // CHUNK


def _pack_body(g_ref, o_ref):
    m = (g_ref[...] > DENSITY_THRESHOLD).astype(jnp.int32)
    sh = lax.broadcasted_iota(jnp.int32, m.shape, 0)
    o_ref[...] = jnp.sum(m << sh, axis=0, keepdims=True)


_PACK_BK = 4096
_pack = pl.pallas_call(
    _pack_body,
    out_shape=jax.ShapeDtypeStruct((1, N_WORDS), jnp.int32),
    grid=(N_WORDS // _PACK_BK,),
    in_specs=[pl.BlockSpec((32, _PACK_BK), lambda i: (0, i))],
    out_specs=pl.BlockSpec((1, _PACK_BK), lambda i: (0, i)),
)


@functools.partial(
    pl.kernel,
    mesh=plsc.VectorSubcoreMesh(core_axis_name="c", subcore_axis_name="s"),
    out_type=jax.ShapeDtypeStruct((N_POINTS,), jnp.int32),
    compiler_params=pltpu.CompilerParams(needs_layout_passes=False),
    scratch_types=[
        pltpu.VMEM((N_WORDS,), jnp.int32),
        pltpu.VMEM((CHUNK * 3,), jnp.float32),
        pltpu.VMEM((CHUNK,), jnp.int32),
    ],
)
def _sc_filter(xyz_hbm, bits_hbm, out_hbm, bits_v, xyz_v, out_v):
    wid = lax.axis_index("s") * 2 + lax.axis_index("c")
    pltpu.sync_copy(bits_hbm, bits_v)
    base = wid * PTS_PER_WORKER
    lane = lax.broadcasted_iota(jnp.int32, (16,), 0)
    lane3 = lane * 3

    def chunk_body(ci, carry):
        start = base + ci * CHUNK
        pltpu.sync_copy(xyz_hbm.at[pl.ds(start * 3, CHUNK * 3)], xyz_v)

        def grp(g, c2):
            o = g * 16
            ix = lane3 + o * 3
            x = plsc.load_gather(xyz_v, [ix])
            y = plsc.load_gather(xyz_v, [ix + 1])
            z = plsc.load_gather(xyz_v, [ix + 2])
            inb = (jnp.abs(x) <= 1.0) & (jnp.abs(y) <= 1.0) & (jnp.abs(z) <= 1.0)

            def vox(v):
                t = jnp.clip((v + 1.0) * 64.0, 0.5, 127.5)
                return t.astype(jnp.int32)

            f = (vox(z) * RES + vox(y)) * RES + vox(x)
            w = f & (N_WORDS - 1)
            b = lax.shift_right_logical(f, 16)
            wv = plsc.load_gather(bits_v, [w])
            hit = (lax.shift_right_logical(wv, b) & 1) != 0
            out_v[pl.ds(o, 16)] = jnp.where(inb & hit, 1, 0).astype(jnp.int32)
            return c2

        lax.fori_loop(0, CHUNK // 16, grp, None)
        pltpu.sync_copy(out_v, out_hbm.at[pl.ds(start, CHUNK)])
        return carry

    lax.fori_loop(0, N_CHUNKS, chunk_body, None)


def kernel(xyz_ndc, grid):
    bits = _pack(grid.reshape(32, N_WORDS)).reshape(N_WORDS)
    out = _sc_filter(xyz_ndc.reshape(-1), bits)
    return out != 0


# speedup vs baseline: 17.4263x; 17.4263x over previous
import functools

import jax
import jax.numpy as jnp
from jax import lax
from jax.experimental import pallas as pl
from jax.experimental.pallas import tpu as pltpu
from jax.experimental.pallas import tpu_sc as plsc

RES = 128
DENSITY_THRESHOLD = 0.01
N_POINTS = 2097152
N_WORDS = RES ** 3 // 32

N_WORKERS = 32
PTS_PER_WORKER = N_POINTS // N_WORKERS
CHUNK = 4096
N_CHUNKS = PTS_PER_WORKER // CHUNK


def _pack_body(g_ref, o_ref):
    m = (g_ref[...] > DENSITY_THRESHOLD).astype(jnp.int32)
    sh = lax.broadcasted_iota(jnp.int32, m.shape, 0)
    o_ref[...] = jnp.sum(m << sh, axis=0, keepdims=True)


_PACK_BK = 4096
_pack = pl.pallas_call(
    _pack_body,
    out_shape=jax.ShapeDtypeStruct((1, N_WORDS), jnp.int32),
    grid=(N_WORDS // _PACK_BK,),
    in_specs=[pl.BlockSpec((32, _PACK_BK), lambda i: (0, i))],
    out_specs=pl.BlockSpec((1, _PACK_BK), lambda i: (0, i)),
)


@functools.partial(
    pl.kernel,
    mesh=plsc.VectorSubcoreMesh(core_axis_name="c", subcore_axis_name="s"),
    out_type=jax.ShapeDtypeStruct((N_POINTS,), jnp.int32),
    compiler_params=pltpu.CompilerParams(needs_layout_passes=False),
    scratch_types=[
        pltpu.VMEM((N_WORDS,), jnp.int32),
        pltpu.VMEM((CHUNK,), jnp.float32),
        pltpu.VMEM((CHUNK,), jnp.float32),
        pltpu.VMEM((CHUNK,), jnp.float32),
        pltpu.VMEM((CHUNK,), jnp.int32),
    ],
)
def _sc_filter(x_hbm, y_hbm, z_hbm, bits_hbm, out_hbm, bits_v, x_v, y_v, z_v, out_v):
    wid = lax.axis_index("s") * 2 + lax.axis_index("c")
    pltpu.sync_copy(bits_hbm, bits_v)
    base = wid * PTS_PER_WORKER

    def chunk_body(ci, carry):
        start = base + ci * CHUNK
        pltpu.sync_copy(x_hbm.at[pl.ds(start, CHUNK)], x_v)
        pltpu.sync_copy(y_hbm.at[pl.ds(start, CHUNK)], y_v)
        pltpu.sync_copy(z_hbm.at[pl.ds(start, CHUNK)], z_v)

        def grp(g, c2):
            o = g * 16
            x = x_v[pl.ds(o, 16)]
            y = y_v[pl.ds(o, 16)]
            z = z_v[pl.ds(o, 16)]
            inb = (jnp.abs(x) <= 1.0) & (jnp.abs(y) <= 1.0) & (jnp.abs(z) <= 1.0)

            def vox(v):
                t = jnp.clip((v + 1.0) * 64.0, 0.5, 127.5)
                return t.astype(jnp.int32)

            f = (vox(z) * RES + vox(y)) * RES + vox(x)
            w = f & (N_WORDS - 1)
            b = lax.shift_right_logical(f, 16)
            wv = plsc.load_gather(bits_v, [w])
            hit = (lax.shift_right_logical(wv, b) & 1) != 0
            out_v[pl.ds(o, 16)] = jnp.where(inb & hit, 1, 0).astype(jnp.int32)
            return c2

        lax.fori_loop(0, CHUNK // 16, grp, None)
        pltpu.sync_copy(out_v, out_hbm.at[pl.ds(start, CHUNK)])
        return carry

    lax.fori_loop(0, N_CHUNKS, chunk_body, None)


def kernel(xyz_ndc, grid):
    bits = _pack(grid.reshape(32, N_WORDS)).reshape(N_WORDS)
    out = _sc_filter(xyz_ndc[:, 0], xyz_ndc[:, 1], xyz_ndc[:, 2], bits)
    return out != 0
